# Initial kernel scaffold; baseline (speedup 1.0000x reference)
import jax
import jax.numpy as jnp
from jax.experimental import pallas as pl


def kernel(x1, edge_index1, batch1, fp1, x2, edge_index2, batch2, fp2, cell, params):
    raise NotImplementedError("write your pallas kernel here")


# speedup vs baseline: 1.0014x; 1.0014x over previous
import functools
import jax
import jax.numpy as jnp
from jax.experimental import pallas as pl
from jax.experimental.pallas import tpu as pltpu

_BLK_B = 256


def _lrelu(x, s=0.01):
    return jnp.where(x > 0, x, s * x)


def _sigmoid(x):
    return 1.0 / (1.0 + jnp.exp(-x))


_BN_SCALE = float((1.0 + 1e-5) ** -0.5)


def _tail_body(x1g, x2g, f1, f2, cv, *refs):
    (out_ref,) = refs[-1:]
    params = refs[:-1]
    it = iter(params)

    feats = [x1g[...], x2g[...], f1[...], f2[...], cv[...]]
    outs = list(feats)
    for s in range(5):
        x = outs[s]
        for li in range(2):
            wg, bg, wn, bn_, wl, bl = (next(it)[...] for _ in range(6))
            g = _sigmoid(jnp.dot(x, wg, preferred_element_type=jnp.float32) + bg)
            nl = _lrelu(jnp.dot(x, wn, preferred_element_type=jnp.float32) + bn_)
            ln = jnp.dot(x, wl, preferred_element_type=jnp.float32) + bl
            x = g * nl + (1.0 - g) * ln + x
        outs[s] = x
    wgp, bgp, ggp, bbgp = (next(it)[...] for _ in range(4))
    gc = jnp.concatenate(outs, axis=1)
    gpo = jnp.dot(gc, wgp, preferred_element_type=jnp.float32) + bgp
    gpo = jnp.maximum(gpo * _BN_SCALE * ggp + bbgp, 0.0)
    w1, b1, w2, b2 = (next(it)[...] for _ in range(4))
    m = jnp.stack([f.mean(axis=1) for f in feats], axis=1)
    w = _sigmoid(jnp.dot(jnp.maximum(jnp.dot(m, w1, preferred_element_type=jnp.float32) + b1, 0.0),
                         w2, preferred_element_type=jnp.float32) + b2)
    se = sum(feats[s] * w[:, s:s + 1] for s in range(5))
    ws1, bs1, ws2, bs2, ws3, bs3 = (next(it)[...] for _ in range(6))
    dual = jnp.concatenate([gpo, se], axis=1)
    h = _lrelu(jnp.dot(dual, ws1, preferred_element_type=jnp.float32) + bs1)
    h = _lrelu(jnp.dot(h, ws2, preferred_element_type=jnp.float32) + bs2)
    out_ref[...] = jnp.dot(h, ws3, preferred_element_type=jnp.float32) + bs3


def _tail(x1g, x2g, f1, f2, cv, params):
    b = x1g.shape[0]
    plist = []
    for s in ['d1g', 'd2g', 'd1f', 'd2f', 'cell']:
        for li in range(2):
            lp = params['mpgr'][s][li]
            plist += [lp['gate']['w'], lp['gate']['b'], lp['nl']['w'], lp['nl']['b'],
                      lp['lin']['w'], lp['lin']['b']]
    gp = params['gp']
    plist += [gp['l']['w'], gp['l']['b'], gp['bn']['g'], gp['bn']['b']]
    se = params['se']
    plist += [se['l1']['w'], se['l1']['b'], se['l2']['w'], se['l2']['b']]
    syn = params['syn']
    plist += [syn['l1']['w'], syn['l1']['b'], syn['l2']['w'], syn['l2']['b'],
              syn['l3']['w'], syn['l3']['b']]

    grid = (b // _BLK_B,)
    feat_spec = pl.BlockSpec((_BLK_B, 128), lambda i: (i, 0))
    pspecs = [pl.BlockSpec(p.shape, lambda i, _r=len(p.shape): (0,) * _r) for p in plist]
    return pl.pallas_call(
        _tail_body,
        grid=grid,
        in_specs=[feat_spec] * 5 + pspecs,
        out_specs=pl.BlockSpec((_BLK_B, 2), lambda i: (i, 0)),
        out_shape=jax.ShapeDtypeStruct((b, 2), jnp.float32),
    )(x1g, x2g, f1, f2, cv, *plist)


def _lin(x, p):
    return x @ p['w'] + p['b']


def _bn(x, p):
    return x * _BN_SCALE * p['g'] + p['b']


def _gat_layer(x, src, dst, p, n):
    h = x @ p['w']
    a = jax.nn.leaky_relu((h * p['as']).sum(-1)[src] + (h * p['ad']).sum(-1)[dst], 0.2)
    amax = jax.ops.segment_max(a, dst, num_segments=n)
    ea = jnp.exp(a - amax[dst])
    den = jax.ops.segment_sum(ea, dst, num_segments=n)
    w = ea / (den[dst] + 1e-16)
    out = jax.ops.segment_sum(h[src] * w[:, None], dst, num_segments=n)
    return out + p['b']


def _fem(x, ei, batch, p, b):
    n = x.shape[0]
    loop = jnp.arange(n, dtype=ei.dtype)
    src = jnp.concatenate([ei[0], loop])
    dst = jnp.concatenate([ei[1], loop])
    for gp in p['gat']:
        x = _lrelu(_gat_layer(x, src, dst, gp, n))
    cnt = jax.ops.segment_sum(jnp.ones((n,), x.dtype), batch, num_segments=b)
    pooled = jax.ops.segment_sum(x, batch, num_segments=b) / jnp.maximum(cnt, 1.0)[:, None]
    h = _lrelu(_bn(_lin(pooled, p['fc1']), p['fc1_bn']))
    return _lin(h, p['fc2'])


def _fp_enc(x, p):
    h = jax.nn.relu(_bn(_lin(x, p['l1']), p['bn1']))
    return jax.nn.relu(_bn(_lin(h, p['l2']), p['bn2']))


def _cell_enc(x, p):
    x = x / jnp.maximum(jnp.linalg.norm(x, axis=1, keepdims=True), 1e-12)
    h = _lrelu(_bn(_lin(x, p['l1']), p['bn1']))
    h = _lrelu(_bn(_lin(h, p['l2']), p['bn2']))
    return _lin(h, p['l3'])


def kernel(x1, edge_index1, batch1, fp1, x2, edge_index2, batch2, fp2, cell, params):
    b = fp1.shape[0]
    x1g = _fem(x1, edge_index1, batch1, params['fem1'], b)
    x2g = _fem(x2, edge_index2, batch2, params['fem2'], b)
    f1 = _fp_enc(fp1, params['fp'])
    f2 = _fp_enc(fp2, params['fp'])
    cv = _cell_enc(cell, params['cell'])
    return _tail(x1g, x2g, f1, f2, cv, params)


# speedup vs baseline: 14.3900x; 14.3694x over previous
import functools
import jax
import jax.numpy as jnp
from jax import lax
from jax.experimental import pallas as pl
from jax.experimental.pallas import tpu as pltpu
from jax.experimental.pallas import tpu_sc as plsc

_BLK_B = 256

_NN = 50000
_E_REAL = 800000
_E_PAD = 800256
_NDEN = 50048
_KB = 128
_B = 1024
_BP = 1032
_NPOOL = 51200


def _lrelu(x, s=0.01):
    return jnp.where(x > 0, x, s * x)


def _sigmoid(x):
    return 1.0 / (1.0 + jnp.exp(-x))


_BN_SCALE = float((1.0 + 1e-5) ** -0.5)


def _tail_body(x1g, x2g, f1, f2, cv, *refs):
    (out_ref,) = refs[-1:]
    params = refs[:-1]
    it = iter(params)

    feats = [x1g[...], x2g[...], f1[...], f2[...], cv[...]]
    outs = list(feats)
    for s in range(5):
        x = outs[s]
        for li in range(2):
            wg, bg, wn, bn_, wl, bl = (next(it)[...] for _ in range(6))
            g = _sigmoid(jnp.dot(x, wg, preferred_element_type=jnp.float32) + bg)
            nl = _lrelu(jnp.dot(x, wn, preferred_element_type=jnp.float32) + bn_)
            ln = jnp.dot(x, wl, preferred_element_type=jnp.float32) + bl
            x = g * nl + (1.0 - g) * ln + x
        outs[s] = x
    wgp, bgp, ggp, bbgp = (next(it)[...] for _ in range(4))
    gc = jnp.concatenate(outs, axis=1)
    gpo = jnp.dot(gc, wgp, preferred_element_type=jnp.float32) + bgp
    gpo = jnp.maximum(gpo * _BN_SCALE * ggp + bbgp, 0.0)
    w1, b1, w2, b2 = (next(it)[...] for _ in range(4))
    m = jnp.stack([f.mean(axis=1) for f in feats], axis=1)
    w = _sigmoid(jnp.dot(jnp.maximum(jnp.dot(m, w1, preferred_element_type=jnp.float32) + b1, 0.0),
                         w2, preferred_element_type=jnp.float32) + b2)
    se = sum(feats[s] * w[:, s:s + 1] for s in range(5))
    ws1, bs1, ws2, bs2, ws3, bs3 = (next(it)[...] for _ in range(6))
    dual = jnp.concatenate([gpo, se], axis=1)
    h = _lrelu(jnp.dot(dual, ws1, preferred_element_type=jnp.float32) + bs1)
    h = _lrelu(jnp.dot(h, ws2, preferred_element_type=jnp.float32) + bs2)
    out_ref[...] = jnp.dot(h, ws3, preferred_element_type=jnp.float32) + bs3


def _tail(x1g, x2g, f1, f2, cv, params):
    b = x1g.shape[0]
    plist = []
    for s in ['d1g', 'd2g', 'd1f', 'd2f', 'cell']:
        for li in range(2):
            lp = params['mpgr'][s][li]
            plist += [lp['gate']['w'], lp['gate']['b'], lp['nl']['w'], lp['nl']['b'],
                      lp['lin']['w'], lp['lin']['b']]
    gp = params['gp']
    plist += [gp['l']['w'], gp['l']['b'], gp['bn']['g'], gp['bn']['b']]
    se = params['se']
    plist += [se['l1']['w'], se['l1']['b'], se['l2']['w'], se['l2']['b']]
    syn = params['syn']
    plist += [syn['l1']['w'], syn['l1']['b'], syn['l2']['w'], syn['l2']['b'],
              syn['l3']['w'], syn['l3']['b']]

    grid = (b // _BLK_B,)
    feat_spec = pl.BlockSpec((_BLK_B, 128), lambda i: (i, 0))
    pspecs = [pl.BlockSpec(p.shape, lambda i, _r=len(p.shape): (0,) * _r) for p in plist]
    return pl.pallas_call(
        _tail_body,
        grid=grid,
        in_specs=[feat_spec] * 5 + pspecs,
        out_specs=pl.BlockSpec((_BLK_B, 2), lambda i: (i, 0)),
        out_shape=jax.ShapeDtypeStruct((b, 2), jnp.float32),
    )(x1g, x2g, f1, f2, cv, *plist)


@functools.lru_cache(maxsize=None)
def _make_edge_aggr(dim):
    S = dim // 32
    split_edges = (S == 1)
    n_out = 2 if split_edges else S
    n_den = 2 if split_edges else 1
    n_workers = 32 if split_edges else 16
    P = _E_PAD // n_workers
    nblk = P // _KB
    tail = P - nblk * _KB
    spc = 1 if split_edges else S // 2
    mesh = plsc.VectorSubcoreMesh(core_axis_name="c", subcore_axis_name="s")
    out_type = tuple([jax.ShapeDtypeStruct((n_out, _NDEN, 32), jnp.float32)] +
                     [jax.ShapeDtypeStruct((_NDEN,), jnp.float32)] * n_den)
    scratch = [
        pltpu.VMEM_SHARED((_NDEN, 32), jnp.float32),
        pltpu.VMEM_SHARED((_NDEN,), jnp.float32),
        pltpu.VMEM((256, 32), jnp.float32),
        pltpu.VMEM((256,), jnp.float32),
        pltpu.VMEM((_KB,), jnp.int32),
        pltpu.VMEM((_KB,), jnp.int32),
        pltpu.VMEM((_KB,), jnp.float32),
        pltpu.VMEM((_KB,), jnp.float32),
        pltpu.VMEM((_KB,), jnp.float32),
        pltpu.VMEM((_KB, 32), jnp.float32),
        pltpu.VMEM((tail,), jnp.int32),
        pltpu.VMEM((tail,), jnp.int32),
        pltpu.VMEM((tail,), jnp.float32),
        pltpu.VMEM((tail,), jnp.float32),
        pltpu.VMEM((tail,), jnp.float32),
        pltpu.VMEM((tail, 32), jnp.float32),
        pltpu.VMEM((16,), jnp.float32),
        pltpu.SemaphoreType.DMA,
    ]

    @functools.partial(pl.kernel, out_type=out_type, mesh=mesh,
                       scratch_types=scratch,
                       compiler_params=pltpu.CompilerParams(
                           use_tc_tiling_on_sc=False))
    def body(src_h, dst_h, as_h, ad_h, m_h, *rest):
        h_refs = rest[:S]
        acc_o = rest[S]
        den_os = rest[S + 1:S + 1 + n_den]
        (acc_sp, den_sp, zrow, zflat, src_b, dst_b, as_b, ad_b, e_b, rows_b,
         src_t, dst_t, as_t, ad_t, e_t, rows_t, m_v, sem) = rest[S + 1 + n_den:]
        c = lax.axis_index("c")
        s = lax.axis_index("s")

        pltpu.sync_copy(m_h, m_v)

        def zfill(i, _):
            zrow[i, pl.ds(0, 16)] = jnp.zeros((16,), jnp.float32)
            zrow[i, pl.ds(16, 16)] = jnp.zeros((16,), jnp.float32)
            return 0
        lax.fori_loop(0, 256, zfill, 0)

        def zfill1(i, _):
            zflat[pl.ds(i * 16, 16)] = jnp.zeros((16,), jnp.float32)
            return 0
        lax.fori_loop(0, 16, zfill1, 0)

        chunks = [(i * 256, 256) for i in range(12)] + [(3072, 56)]

        def _stripe(off, sz):
            return pl.multiple_of(s * 3128 + off, 8), sz

        def zero_acc():
            for off, sz in chunks:
                o, _ = _stripe(off, sz)
                pltpu.sync_copy(zrow.at[pl.ds(0, sz)],
                                acc_sp.at[pl.ds(o, sz)])

        def zero_den():
            for off, sz in chunks:
                o, _ = _stripe(off, sz)
                pltpu.sync_copy(zflat.at[pl.ds(0, sz)],
                                den_sp.at[pl.ds(o, sz)])

        def copy_acc(out_idx):
            for off, sz in chunks:
                o, _ = _stripe(off, sz)
                pltpu.sync_copy(acc_sp.at[pl.ds(o, sz)],
                                acc_o.at[out_idx, pl.ds(o, sz)])

        def copy_den(den_idx):
            for off, sz in chunks:
                o, _ = _stripe(off, sz)
                pltpu.sync_copy(den_sp.at[pl.ds(o, sz)],
                                den_os[den_idx].at[pl.ds(o, sz)])

        def process(base, kk, sb, db, ab, bb, eb, rb, hsl_ref, with_den):
            base = pl.multiple_of(base, 8)
            pltpu.sync_copy(src_h.at[pl.ds(base, kk)], sb)
            pltpu.sync_copy(dst_h.at[pl.ds(base, kk)], db)
            pltpu.async_copy(as_h.at[sb], ab, sem).wait()
            pltpu.async_copy(ad_h.at[db], bb, sem).wait()
            m = m_v[pl.ds(0, 16)][0]
            for j in range(kk // 16):
                z = ab[pl.ds(j * 16, 16)] + bb[pl.ds(j * 16, 16)]
                a = jnp.maximum(z, 0.0) + 0.2 * jnp.minimum(z, 0.0)
                e = jnp.exp(a - m)
                gidx = (base + j * 16) + lax.iota(jnp.int32, 16)
                e = jnp.where(gidx < _E_REAL, e, jnp.zeros((16,), jnp.float32))
                eb[pl.ds(j * 16, 16)] = e
            if with_den:
                pltpu.sync_copy(eb, den_sp.at[db], add=True)
            pltpu.async_copy(hsl_ref.at[sb], rb, sem).wait()

            def scale(j, _):
                ev = eb[pl.ds(j * 16, 16)]
                for k2 in range(16):
                    r = j * 16 + k2
                    ek = ev[k2]
                    rb[r, pl.ds(0, 16)] = rb[r, pl.ds(0, 16)] * ek
                    rb[r, pl.ds(16, 16)] = rb[r, pl.ds(16, 16)] * ek
                return 0
            lax.fori_loop(0, kk // 16, scale, 0)
            pltpu.sync_copy(rb, acc_sp.at[db], add=True)

        def run_pass(hsl_ref, out_idx, with_den, den_idx, base0):
            zero_acc()
            if with_den:
                zero_den()
            plsc.subcore_barrier()

            def blk(i, _):
                process(base0 + i * _KB, _KB, src_b, dst_b, as_b, ad_b, e_b,
                        rows_b, hsl_ref, with_den)
                return 0
            lax.fori_loop(0, nblk, blk, 0)
            if tail:
                process(base0 + nblk * _KB, tail, src_t, dst_t, as_t, ad_t,
                        e_t, rows_t, hsl_ref, with_den)
            plsc.subcore_barrier()
            copy_acc(out_idx)
            if with_den:
                copy_den(den_idx)
            plsc.subcore_barrier()

        for cc in range(2):
            @pl.when(c == cc)
            def _(cc=cc):
                if split_edges:
                    run_pass(h_refs[0], out_idx=cc, with_den=True, den_idx=cc,
                             base0=(s * 2 + cc) * P)
                else:
                    for si in range(spc):
                        k_idx = cc * spc + si
                        run_pass(h_refs[k_idx], out_idx=k_idx,
                                 with_den=(cc == 0 and si == 0), den_idx=0,
                                 base0=s * P)

    return body


@functools.lru_cache(maxsize=None)
def _make_pool():
    rows_per_w = _NPOOL // 32
    nblk = rows_per_w // _KB
    tail = rows_per_w - nblk * _KB
    mesh = plsc.VectorSubcoreMesh(core_axis_name="c", subcore_axis_name="s")
    out_type = (jax.ShapeDtypeStruct((2, _BP, 128), jnp.float32),
                jax.ShapeDtypeStruct((_BP,), jnp.float32),
                jax.ShapeDtypeStruct((_BP,), jnp.float32))
    scratch = [
        pltpu.VMEM_SHARED((_BP, 128), jnp.float32),
        pltpu.VMEM_SHARED((_BP,), jnp.float32),
        pltpu.VMEM((64, 128), jnp.float32),
        pltpu.VMEM((256,), jnp.float32),
        pltpu.VMEM((_KB, 128), jnp.float32),
        pltpu.VMEM((_KB,), jnp.int32),
        pltpu.VMEM((_KB,), jnp.float32),
        pltpu.VMEM((tail, 128), jnp.float32),
        pltpu.VMEM((tail,), jnp.int32),
        pltpu.VMEM((tail,), jnp.float32),
        pltpu.SemaphoreType.DMA,
    ]

    @functools.partial(pl.kernel, out_type=out_type, mesh=mesh,
                       scratch_types=scratch)
    def body(x_h, b_h, acc_o, cnt_o0, cnt_o1, acc_sp, cnt_sp, zp, zf, rows,
             bidx, ones_v, rows_t, bidx_t, ones_t, sem):
        cnt_os = (cnt_o0, cnt_o1)
        c = lax.axis_index("c")
        s = lax.axis_index("s")

        def zfill(i, _):
            for j in range(8):
                zp[i, pl.ds(j * 16, 16)] = jnp.zeros((16,), jnp.float32)
            return 0
        lax.fori_loop(0, 64, zfill, 0)

        def zfill1(i, _):
            zf[pl.ds(i * 16, 16)] = jnp.zeros((16,), jnp.float32)
            return 0
        lax.fori_loop(0, 16, zfill1, 0)

        for j in range(_KB // 16):
            ones_v[pl.ds(j * 16, 16)] = jnp.ones((16,), jnp.float32)
        for j in range(tail // 16):
            ones_t[pl.ds(j * 16, 16)] = jnp.ones((16,), jnp.float32)

        @pl.when(s == 0)
        def _():
            for i in range(16):
                pltpu.sync_copy(zp, acc_sp.at[pl.ds(i * 64, 64)])
            pltpu.sync_copy(zp.at[pl.ds(0, 8)], acc_sp.at[pl.ds(1024, 8)])
            for i in range(4):
                pltpu.sync_copy(zf, cnt_sp.at[pl.ds(i * 256, 256)])
            pltpu.sync_copy(zf.at[pl.ds(0, 8)], cnt_sp.at[pl.ds(1024, 8)])
        plsc.subcore_barrier()

        def do_block(rb, kk, rbuf, ibuf, obuf):
            rb = pl.multiple_of(rb, 8)
            pltpu.sync_copy(x_h.at[pl.ds(rb, kk)], rbuf)
            pltpu.sync_copy(b_h.at[pl.ds(rb, kk)], ibuf)
            pltpu.sync_copy(rbuf, acc_sp.at[ibuf], add=True)
            pltpu.sync_copy(obuf, cnt_sp.at[ibuf], add=True)

        for cc in range(2):
            @pl.when(c == cc)
            def _(cc=cc):
                base0 = (s * 2 + cc) * rows_per_w

                def blk(i, _):
                    do_block(base0 + i * _KB, _KB, rows, bidx, ones_v)
                    return 0
                lax.fori_loop(0, nblk, blk, 0)
                do_block(base0 + nblk * _KB, tail, rows_t, bidx_t, ones_t)
                plsc.subcore_barrier()

                @pl.when(s == 0)
                def _():
                    pltpu.sync_copy(acc_sp, acc_o.at[cc])
                    pltpu.sync_copy(cnt_sp, cnt_os[cc])

    return body


def _lin(x, p):
    return x @ p['w'] + p['b']


def _bn(x, p):
    return x * _BN_SCALE * p['g'] + p['b']


def _gat_layer_sc(x, src_p, dst_p, p):
    h = x @ p['w']
    dim = h.shape[1]
    S = dim // 32
    asum = h @ p['as']
    adsum = h @ p['ad']
    mb = jnp.max(asum) + jnp.max(adsum)
    m = jnp.where(mb > 0, mb, 0.2 * mb)
    marr = jnp.full((16,), m, jnp.float32)
    hsl = [h[:, 32 * i:32 * (i + 1)] for i in range(S)]
    res = _make_edge_aggr(dim)(src_p, dst_p, asum, adsum, marr, *hsl)
    acc_p, den_p = res[0], res[1:]
    if S == 1:
        acc = acc_p[0] + acc_p[1]
        den = den_p[0] + den_p[1]
    else:
        acc = jnp.concatenate([acc_p[i] for i in range(S)], axis=1)
        den = den_p[0]
    acc = acc[:_NN]
    den = den[:_NN]
    z = asum + adsum
    e_self = jnp.exp(jnp.where(z > 0, z, 0.2 * z) - m)
    num = acc + e_self[:, None] * h
    dtot = den + e_self + 1e-16
    return num / dtot[:, None] + p['b']


def _fem(x, ei, batch, p, b):
    src_p = jnp.pad(ei[0], (0, _E_PAD - _E_REAL))
    dst_p = jnp.pad(ei[1], (0, _E_PAD - _E_REAL))
    for gp in p['gat']:
        x = _lrelu(_gat_layer_sc(x, src_p, dst_p, gp))
    xp = jnp.pad(x, ((0, _NPOOL - _NN), (0, 0)))
    bp = jnp.pad(batch, (0, _NPOOL - _NN), constant_values=_B)
    acc_p, cnt0, cnt1 = _make_pool()(xp, bp)
    pooled_sum = (acc_p[0] + acc_p[1])[:_B]
    cnt = (cnt0 + cnt1)[:_B]
    pooled = pooled_sum / jnp.maximum(cnt, 1.0)[:, None]
    h = _lrelu(_bn(_lin(pooled, p['fc1']), p['fc1_bn']))
    return _lin(h, p['fc2'])


def _fp_enc(x, p):
    h = jax.nn.relu(_bn(_lin(x, p['l1']), p['bn1']))
    return jax.nn.relu(_bn(_lin(h, p['l2']), p['bn2']))


def _cell_enc(x, p):
    x = x / jnp.maximum(jnp.linalg.norm(x, axis=1, keepdims=True), 1e-12)
    h = _lrelu(_bn(_lin(x, p['l1']), p['bn1']))
    h = _lrelu(_bn(_lin(h, p['l2']), p['bn2']))
    return _lin(h, p['l3'])


def kernel(x1, edge_index1, batch1, fp1, x2, edge_index2, batch2, fp2, cell, params):
    b = fp1.shape[0]
    x1g = _fem(x1, edge_index1, batch1, params['fem1'], b)
    x2g = _fem(x2, edge_index2, batch2, params['fem2'], b)
    f1 = _fp_enc(fp1, params['fp'])
    f2 = _fp_enc(fp2, params['fp'])
    cv = _cell_enc(cell, params['cell'])
    return _tail(x1g, x2g, f1, f2, cv, params)
